# baseline (speedup 1.0000x reference)
import functools

import jax
import jax.numpy as jnp
from jax import lax
from jax.experimental import pallas as pl
from jax.experimental.pallas import tpu as pltpu
from jax.experimental.pallas import tpu_sc as plsc

D = 64
NC, NS = 2, 16
NW = NC * NS
GROUP = 128
NGROUPS = 200


def _gather_body(table_hbm, idx_hbm, out_hbm, idx_v, rows_v, gsem):
    c = lax.axis_index("c")
    s = lax.axis_index("s")
    wid = s * NC + c

    pltpu.sync_copy(idx_hbm.at[wid], idx_v)

    def step(j, carry):
        buf = lax.rem(j, 2)
        pltpu.async_copy(table_hbm.at[idx_v.at[j]], rows_v.at[buf], gsem).wait()
        pltpu.sync_copy(rows_v.at[buf], out_hbm.at[wid].at[j])
        return carry

    lax.fori_loop(0, NGROUPS, step, 0)


@functools.partial(jax.jit, static_argnames=())
def kernel(input_words, table):
    nrows, seq = input_words.shape
    total = nrows * seq
    idx = input_words.reshape(NW, total // (NW * GROUP), GROUP).astype(jnp.int32)

    mesh = plsc.VectorSubcoreMesh(core_axis_name="c", subcore_axis_name="s")
    out = pl.kernel(
        _gather_body,
        out_type=jax.ShapeDtypeStruct((NW, total // (NW * GROUP), GROUP, D),
                                      jnp.float32),
        mesh=mesh,
        scratch_types=[
            pltpu.VMEM((NGROUPS, GROUP), jnp.int32),
            pltpu.VMEM((2, GROUP, D), jnp.float32),
            pltpu.SemaphoreType.DMA,
        ],
        compiler_params=pltpu.CompilerParams(use_tc_tiling_on_sc=False),
    )(table, idx)
    return out.reshape(nrows, seq, D)


# speedup vs baseline: 1.1127x; 1.1127x over previous
import functools

import jax
import jax.numpy as jnp
from jax import lax
from jax.experimental import pallas as pl
from jax.experimental.pallas import tpu as pltpu
from jax.experimental.pallas import tpu_sc as plsc

D = 64
NC, NS = 2, 16
NW = NC * NS
GROUP = 128
NGROUPS = 200


S = 5
NB = 2
NCHUNK = NGROUPS // S


def _gather_body(table_hbm, idx_hbm, out_hbm, idx_v, rows_v, gsem, ssem0, ssem1):
    c = lax.axis_index("c")
    s = lax.axis_index("s")
    wid = s * NC + c
    ssems = (ssem0, ssem1)

    pltpu.sync_copy(idx_hbm.at[wid], idx_v)

    @pl.loop(0, NCHUNK, step=NB)
    def outer(t):
        for b in range(NB):
            tt = t + b

            @pl.when(tt >= NB)
            def _():
                pltpu.make_async_copy(
                    rows_v.at[b],
                    out_hbm.at[wid].at[pl.ds((tt - NB) * S, S)],
                    ssems[b],
                ).wait()

            descs = [
                pltpu.async_copy(
                    table_hbm.at[idx_v.at[tt * S + i]],
                    rows_v.at[b].at[i],
                    gsem,
                )
                for i in range(S)
            ]
            for d in descs:
                d.wait()

            pltpu.async_copy(
                rows_v.at[b],
                out_hbm.at[wid].at[pl.ds(tt * S, S)],
                ssems[b],
            )

    for b in range(NB):
        last = NCHUNK - NB + b
        pltpu.make_async_copy(
            rows_v.at[b],
            out_hbm.at[wid].at[pl.ds(last * S, S)],
            ssems[b],
        ).wait()


@functools.partial(jax.jit, static_argnames=())
def kernel(input_words, table):
    nrows, seq = input_words.shape
    total = nrows * seq
    idx = input_words.reshape(NW, total // (NW * GROUP), GROUP).astype(jnp.int32)

    mesh = plsc.VectorSubcoreMesh(core_axis_name="c", subcore_axis_name="s")
    out = pl.kernel(
        _gather_body,
        out_type=jax.ShapeDtypeStruct((NW, total // (NW * GROUP), GROUP, D),
                                      jnp.float32),
        mesh=mesh,
        scratch_types=[
            pltpu.VMEM((NGROUPS, GROUP), jnp.int32),
            pltpu.VMEM((NB, S, GROUP, D), jnp.float32),
            pltpu.SemaphoreType.DMA,
            pltpu.SemaphoreType.DMA,
            pltpu.SemaphoreType.DMA,
        ],
        compiler_params=pltpu.CompilerParams(use_tc_tiling_on_sc=False),
    )(table, idx)
    return out.reshape(nrows, seq, D)


# speedup vs baseline: 1.1132x; 1.0005x over previous
import functools

import jax
import jax.numpy as jnp
from jax import lax
from jax.experimental import pallas as pl
from jax.experimental.pallas import tpu as pltpu
from jax.experimental.pallas import tpu_sc as plsc

D = 64
NC, NS = 2, 16
NW = NC * NS
SENT_PER_W = 128
SEQ = 200
SPLIT = (0, 96, SEQ)
S = 4
NB = 2
NCHUNK = SENT_PER_W // S


def _gather_body(table_hbm, idx_hbm, out_hbm, idx_v, rows_v, gsem, ssem0, ssem1):
    c = lax.axis_index("c")
    s = lax.axis_index("s")
    wid = s * NC + c
    sent0 = wid * SENT_PER_W
    ssems = (ssem0, ssem1)

    pltpu.sync_copy(idx_hbm.at[pl.ds(sent0, SENT_PER_W)], idx_v)

    @pl.loop(0, NCHUNK, step=NB)
    def outer(t):
        for b in range(NB):
            tt = t + b

            @pl.when(tt >= NB)
            def _():
                pltpu.make_async_copy(
                    rows_v.at[b],
                    out_hbm.at[pl.ds(sent0 + (tt - NB) * S, S)],
                    ssems[b],
                ).wait()

            descs = []
            for i in range(S):
                row = idx_v.at[tt * S + i]
                dst = rows_v.at[b].at[i]
                for k in range(len(SPLIT) - 1):
                    lo, hi = SPLIT[k], SPLIT[k + 1]
                    descs.append(pltpu.async_copy(
                        table_hbm.at[row.at[pl.ds(lo, hi - lo)]],
                        dst.at[pl.ds(lo, hi - lo)],
                        gsem,
                    ))
            for d in descs:
                d.wait()

            pltpu.async_copy(
                rows_v.at[b],
                out_hbm.at[pl.ds(sent0 + tt * S, S)],
                ssems[b],
            )

    for b in range(NB):
        last = NCHUNK - NB + b
        pltpu.make_async_copy(
            rows_v.at[b],
            out_hbm.at[pl.ds(sent0 + last * S, S)],
            ssems[b],
        ).wait()


@jax.jit
def kernel(input_words, table):
    nsent, seq = input_words.shape
    idx = input_words.astype(jnp.int32)

    mesh = plsc.VectorSubcoreMesh(core_axis_name="c", subcore_axis_name="s")
    return pl.kernel(
        _gather_body,
        out_type=jax.ShapeDtypeStruct((nsent, seq, D), jnp.float32),
        mesh=mesh,
        scratch_types=[
            pltpu.VMEM((SENT_PER_W, SEQ), jnp.int32),
            pltpu.VMEM((NB, S, SEQ, D), jnp.float32),
            pltpu.SemaphoreType.DMA,
            pltpu.SemaphoreType.DMA,
            pltpu.SemaphoreType.DMA,
        ],
        compiler_params=pltpu.CompilerParams(use_tc_tiling_on_sc=False),
    )(table, idx)
